# baseline (speedup 1.0000x reference)
import jax
import jax.numpy as jnp
from jax import lax
from jax.experimental import pallas as pl
from jax.experimental.pallas import tpu as pltpu
from jax.experimental.pallas import tpu_sc as plsc

VOCAB = 100000
POS = 514
D = 128
B = 1024
L = 200
N = B * L

NUM_CORES = 2
NUM_SUBCORES = 16
NW = NUM_CORES * NUM_SUBCORES
N_PER_W = N // NW
CB = 128
NCH = N_PER_W // CB
NSTAGE = 4


def _body(act_hbm, st_hbm, pos_hbm, at_hbm, stt_hbm, pt_hbm, out_hbm,
          ia, isx, ip, pt_sh,
          o0, o1, o2, o3,
          sa0, ss0, sp0, so0, sa1, ss1, sp1, so1,
          sa2, ss2, sp2, so2, sa3, ss3, sp3, so3):
    sid = lax.axis_index("s")
    wid = sid * NUM_CORES + lax.axis_index("c")
    base = wid * N_PER_W

    @pl.when(sid == 0)
    def _():
        pltpu.sync_copy(pt_hbm, pt_sh)

    pltpu.sync_copy(act_hbm.at[wid], ia)
    pltpu.sync_copy(st_hbm.at[wid], isx)
    pltpu.sync_copy(pos_hbm.at[wid], ip)
    plsc.subcore_barrier()

    stages = (
        (o0, sa0, ss0, sp0, so0),
        (o1, sa1, ss1, sp1, so1),
        (o2, sa2, ss2, sp2, so2),
        (o3, sa3, ss3, sp3, so3),
    )

    def out_slice(c):
        return out_hbm.at[pl.ds(base + c * CB, CB)]

    def chunk_step(c, st):
        o_v, sa, ss, sp, so = st

        @pl.when(c >= NSTAGE)
        def _():
            pltpu.make_async_copy(o_v, out_slice(c - NSTAGE), so).wait()

        pltpu.async_copy(at_hbm.at[ia.at[c]], o_v, sa)
        pltpu.make_async_copy(at_hbm.at[ia.at[c]], o_v, sa).wait()
        pltpu.async_copy(stt_hbm.at[isx.at[c]], o_v, ss, add=True)
        pltpu.make_async_copy(stt_hbm.at[isx.at[c]], o_v, ss).wait()
        pltpu.async_copy(pt_sh.at[ip.at[c]], o_v, sp, add=True)
        pltpu.make_async_copy(pt_sh.at[ip.at[c]], o_v, sp).wait()
        pltpu.async_copy(o_v, out_slice(c), so)

    def body(cc, _):
        for s in range(NSTAGE):
            chunk_step(cc * NSTAGE + s, stages[s])
        return _

    n_full = NCH // NSTAGE
    lax.fori_loop(0, n_full, body, None)
    for c in range(n_full * NSTAGE, NCH):
        chunk_step(c, stages[c % NSTAGE])

    for c in range(NCH - NSTAGE, NCH):
        o_v, _, _, _, so = stages[c % NSTAGE]
        pltpu.make_async_copy(o_v, out_slice(c), so).wait()


@jax.jit
def _run(actions_idx, states_idx, positions_idx,
         action_table, state_table, pos_table):
    mesh = plsc.VectorSubcoreMesh(core_axis_name="c", subcore_axis_name="s")
    f = pl.kernel(
        _body,
        out_type=jax.ShapeDtypeStruct((N, D), jnp.float32),
        mesh=mesh,
        scratch_types=(
            [pltpu.VMEM((NCH, CB), jnp.int32)] * 3
            + [pltpu.VMEM_SHARED((POS, D), jnp.float32)]
            + [pltpu.VMEM((CB, D), jnp.float32)] * NSTAGE
            + [pltpu.SemaphoreType.DMA] * (4 * NSTAGE)
        ),
    )
    return f(actions_idx, states_idx, positions_idx,
             action_table, state_table, pos_table)


def kernel(actions, states, positions, action_table, state_table, pos_table):
    a = actions.reshape(NW, NCH, CB).astype(jnp.int32)
    s = states.reshape(NW, NCH, CB).astype(jnp.int32)
    p = positions.reshape(NW, NCH, CB).astype(jnp.int32)
    out = _run(a, s, p, action_table, state_table, pos_table)
    return out.reshape(B, L, D)


# speedup vs baseline: 1.5269x; 1.5269x over previous
import jax
import jax.numpy as jnp
from jax import lax
from jax.experimental import pallas as pl
from jax.experimental.pallas import tpu as pltpu
from jax.experimental.pallas import tpu_sc as plsc

VOCAB = 100000
POS = 514
D = 128
B = 1024
L = 200
N = B * L

NUM_CORES = 2
NUM_SUBCORES = 16
NW = NUM_CORES * NUM_SUBCORES
N_PER_W = N // NW
CB = 128
NCH = N_PER_W // CB
NSTAGE = 4


def _body(act_hbm, st_hbm, pos_hbm, at_hbm, stt_hbm, pt_hbm, out_hbm,
          ia, isx, ip, pt_sh,
          o0, o1, o2, o3,
          sa0, ss0, sp0, so0, sa1, ss1, sp1, so1,
          sa2, ss2, sp2, so2, sa3, ss3, sp3, so3):
    sid = lax.axis_index("s")
    wid = sid * NUM_CORES + lax.axis_index("c")
    base = wid * N_PER_W

    @pl.when(sid == 0)
    def _():
        pltpu.sync_copy(pt_hbm, pt_sh)

    pltpu.sync_copy(act_hbm.at[wid], ia)
    pltpu.sync_copy(st_hbm.at[wid], isx)
    pltpu.sync_copy(pos_hbm.at[wid], ip)
    plsc.subcore_barrier()

    stages = (
        (o0, sa0, ss0, sp0, so0),
        (o1, sa1, ss1, sp1, so1),
        (o2, sa2, ss2, sp2, so2),
        (o3, sa3, ss3, sp3, so3),
    )

    def out_slice(c):
        return out_hbm.at[pl.ds(base + c * CB, CB)]

    def phase_step(i):
        if i < NCH:
            o_v, sa, _, _, so = stages[i % NSTAGE]
            if i >= NSTAGE:
                pltpu.make_async_copy(o_v, out_slice(i - NSTAGE), so).wait()
            pltpu.async_copy(at_hbm.at[ia.at[i]], o_v, sa)

        if 1 <= i < NCH + 1:
            c = i - 1
            o_v, sa, ss, _, _ = stages[c % NSTAGE]
            pltpu.make_async_copy(at_hbm.at[ia.at[c]], o_v, sa).wait()
            pltpu.async_copy(stt_hbm.at[isx.at[c]], o_v, ss, add=True)

        if i >= 2:
            c = i - 2
            o_v, _, ss, sp, so = stages[c % NSTAGE]
            pltpu.make_async_copy(stt_hbm.at[isx.at[c]], o_v, ss).wait()
            pltpu.async_copy(pt_sh.at[ip.at[c]], o_v, sp, add=True)
            pltpu.make_async_copy(pt_sh.at[ip.at[c]], o_v, sp).wait()
            pltpu.async_copy(o_v, out_slice(c), so)

    def steady_step(i, k):
        o_a, sa_a, _, _, so_a = stages[k]
        pltpu.make_async_copy(o_a, out_slice(i - NSTAGE), so_a).wait()
        pltpu.async_copy(at_hbm.at[ia.at[i]], o_a, sa_a)

        c1 = i - 1
        o_s, sa_s, ss_s, _, _ = stages[(k - 1) % NSTAGE]
        pltpu.make_async_copy(at_hbm.at[ia.at[c1]], o_s, sa_s).wait()
        pltpu.async_copy(stt_hbm.at[isx.at[c1]], o_s, ss_s, add=True)

        c2 = i - 2
        o_p, _, ss_p, sp_p, so_p = stages[(k - 2) % NSTAGE]
        pltpu.make_async_copy(stt_hbm.at[isx.at[c2]], o_p, ss_p).wait()
        pltpu.async_copy(pt_sh.at[ip.at[c2]], o_p, sp_p, add=True)
        pltpu.make_async_copy(pt_sh.at[ip.at[c2]], o_p, sp_p).wait()
        pltpu.async_copy(o_p, out_slice(c2), so_p)

    PRO = 6
    assert (NCH - PRO) % NSTAGE == 0
    for i in range(PRO):
        phase_step(i)

    def body(ii, _):
        i0 = PRO + ii * NSTAGE
        for s in range(NSTAGE):
            steady_step(i0 + s, (PRO + s) % NSTAGE)
        return _

    lax.fori_loop(0, (NCH - PRO) // NSTAGE, body, None)
    for i in range(NCH, NCH + 2):
        phase_step(i)

    for c in range(NCH - NSTAGE, NCH):
        o_v, _, _, _, so = stages[c % NSTAGE]
        pltpu.make_async_copy(o_v, out_slice(c), so).wait()


@jax.jit
def _run(actions_idx, states_idx, positions_idx,
         action_table, state_table, pos_table):
    mesh = plsc.VectorSubcoreMesh(core_axis_name="c", subcore_axis_name="s")
    f = pl.kernel(
        _body,
        out_type=jax.ShapeDtypeStruct((N, D), jnp.float32),
        mesh=mesh,
        scratch_types=(
            [pltpu.VMEM((NCH, CB), jnp.int32)] * 3
            + [pltpu.VMEM_SHARED((POS, D), jnp.float32)]
            + [pltpu.VMEM((CB, D), jnp.float32)] * NSTAGE
            + [pltpu.SemaphoreType.DMA] * (4 * NSTAGE)
        ),
    )
    return f(actions_idx, states_idx, positions_idx,
             action_table, state_table, pos_table)


def kernel(actions, states, positions, action_table, state_table, pos_table):
    a = actions.reshape(NW, NCH, CB).astype(jnp.int32)
    s = states.reshape(NW, NCH, CB).astype(jnp.int32)
    p = positions.reshape(NW, NCH, CB).astype(jnp.int32)
    out = _run(a, s, p, action_table, state_table, pos_table)
    return out.reshape(B, L, D)


# speedup vs baseline: 1.5332x; 1.0041x over previous
import jax
import jax.numpy as jnp
from jax import lax
from jax.experimental import pallas as pl
from jax.experimental.pallas import tpu as pltpu
from jax.experimental.pallas import tpu_sc as plsc

VOCAB = 100000
POS = 514
D = 128
B = 1024
L = 200
N = B * L

NUM_CORES = 2
NUM_SUBCORES = 16
NW = NUM_CORES * NUM_SUBCORES
N_PER_W = N // NW
CB = 128
NCH = N_PER_W // CB
NSTAGE = 5


def _body(act_hbm, st_hbm, pos_hbm, at_hbm, stt_hbm, pt_hbm, out_hbm,
          ia, isx, ip, pt_sh,
          o0, o1, o2, o3, o4,
          sa0, ss0, sp0, so0, sa1, ss1, sp1, so1,
          sa2, ss2, sp2, so2, sa3, ss3, sp3, so3,
          sa4, ss4, sp4, so4):
    sid = lax.axis_index("s")
    wid = sid * NUM_CORES + lax.axis_index("c")
    base = wid * N_PER_W

    @pl.when(sid == 0)
    def _():
        pltpu.sync_copy(pt_hbm, pt_sh)

    pltpu.sync_copy(act_hbm.at[wid], ia)
    pltpu.sync_copy(st_hbm.at[wid], isx)
    pltpu.sync_copy(pos_hbm.at[wid], ip)
    plsc.subcore_barrier()

    stages = (
        (o0, sa0, ss0, sp0, so0),
        (o1, sa1, ss1, sp1, so1),
        (o2, sa2, ss2, sp2, so2),
        (o3, sa3, ss3, sp3, so3),
        (o4, sa4, ss4, sp4, so4),
    )

    def out_slice(c):
        return out_hbm.at[pl.ds(base + c * CB, CB)]

    def phase_step(i):
        if i < NCH:
            o_v, sa, _, _, so = stages[i % NSTAGE]
            if i >= NSTAGE:
                pltpu.make_async_copy(o_v, out_slice(i - NSTAGE), so).wait()
            pltpu.async_copy(at_hbm.at[ia.at[i]], o_v, sa)

        if 1 <= i < NCH + 1:
            c = i - 1
            o_v, sa, ss, _, _ = stages[c % NSTAGE]
            pltpu.make_async_copy(at_hbm.at[ia.at[c]], o_v, sa).wait()
            pltpu.async_copy(stt_hbm.at[isx.at[c]], o_v, ss, add=True)

        if 2 <= i < NCH + 2:
            c = i - 2
            o_v, _, ss, sp, _ = stages[c % NSTAGE]
            pltpu.make_async_copy(stt_hbm.at[isx.at[c]], o_v, ss).wait()
            pltpu.async_copy(pt_sh.at[ip.at[c]], o_v, sp, add=True)

        if i >= 3:
            c = i - 3
            o_v, _, _, sp, so = stages[c % NSTAGE]
            pltpu.make_async_copy(pt_sh.at[ip.at[c]], o_v, sp).wait()
            pltpu.async_copy(o_v, out_slice(c), so)

    def steady_step(i, k):
        o_a, sa_a, _, _, so_a = stages[k]
        pltpu.make_async_copy(o_a, out_slice(i - NSTAGE), so_a).wait()
        pltpu.async_copy(at_hbm.at[ia.at[i]], o_a, sa_a)

        c1 = i - 1
        o_s, sa_s, ss_s, _, _ = stages[(k - 1) % NSTAGE]
        pltpu.make_async_copy(at_hbm.at[ia.at[c1]], o_s, sa_s).wait()
        pltpu.async_copy(stt_hbm.at[isx.at[c1]], o_s, ss_s, add=True)

        c2 = i - 2
        o_p, _, ss_p, sp_p, _ = stages[(k - 2) % NSTAGE]
        pltpu.make_async_copy(stt_hbm.at[isx.at[c2]], o_p, ss_p).wait()
        pltpu.async_copy(pt_sh.at[ip.at[c2]], o_p, sp_p, add=True)

        c3 = i - 3
        o_o, _, _, sp_o, so_o = stages[(k - 3) % NSTAGE]
        pltpu.make_async_copy(pt_sh.at[ip.at[c3]], o_o, sp_o).wait()
        pltpu.async_copy(o_o, out_slice(c3), so_o)

    PRO = 10
    assert (NCH - PRO) % NSTAGE == 0
    for i in range(PRO):
        phase_step(i)

    def body(ii, _):
        i0 = PRO + ii * NSTAGE
        for s in range(NSTAGE):
            steady_step(i0 + s, (PRO + s) % NSTAGE)
        return _

    lax.fori_loop(0, (NCH - PRO) // NSTAGE, body, None)
    for i in range(NCH, NCH + 3):
        phase_step(i)

    for c in range(NCH - NSTAGE, NCH):
        o_v, _, _, _, so = stages[c % NSTAGE]
        pltpu.make_async_copy(o_v, out_slice(c), so).wait()


@jax.jit
def _run(actions_idx, states_idx, positions_idx,
         action_table, state_table, pos_table):
    mesh = plsc.VectorSubcoreMesh(core_axis_name="c", subcore_axis_name="s")
    f = pl.kernel(
        _body,
        out_type=jax.ShapeDtypeStruct((N, D), jnp.float32),
        mesh=mesh,
        scratch_types=(
            [pltpu.VMEM((NCH, CB), jnp.int32)] * 3
            + [pltpu.VMEM_SHARED((POS, D), jnp.float32)]
            + [pltpu.VMEM((CB, D), jnp.float32)] * NSTAGE
            + [pltpu.SemaphoreType.DMA] * (4 * NSTAGE)
        ),
    )
    return f(actions_idx, states_idx, positions_idx,
             action_table, state_table, pos_table)


def kernel(actions, states, positions, action_table, state_table, pos_table):
    a = actions.reshape(NW, NCH, CB).astype(jnp.int32)
    s = states.reshape(NW, NCH, CB).astype(jnp.int32)
    p = positions.reshape(NW, NCH, CB).astype(jnp.int32)
    out = _run(a, s, p, action_table, state_table, pos_table)
    return out.reshape(B, L, D)
